# baseline (speedup 1.0000x reference)
import functools

import jax
import jax.numpy as jnp
from jax import lax
from jax.experimental import pallas as pl
from jax.experimental.pallas import tpu as pltpu
from jax.experimental.pallas import tpu_sc as plsc

B, L, V, D = 16384, 64, 1000000, 8
BL = B * L
H = L * D
NSLAB = B // 8


_NC, _NS = 2, 16
_NW = _NC * _NS
_PER_W = BL // _NW
_CH = 2048
_NCH = _PER_W // _CH

_C = 2000
_NCHUNK = V // _C
_CHUNK_ITERS = -(-_NCHUNK // _NW)
_BPW = B // _NW
_IDX_CH = 2048


def _sc_prep(tableT_flat, xT_flat):
    mesh = plsc.VectorSubcoreMesh(core_axis_name="c", subcore_axis_name="s")

    @functools.partial(
        pl.kernel,
        mesh=mesh,
        compiler_params=pltpu.CompilerParams(use_tc_tiling_on_sc=False,
                                             needs_layout_passes=False),
        out_type=(jax.ShapeDtypeStruct((V * D,), jnp.float32),
                  jax.ShapeDtypeStruct((BL,), jnp.int32)),
        scratch_types=[
            pltpu.VMEM((D, _C), jnp.float32),
            pltpu.VMEM((_C * D,), jnp.float32),
            pltpu.VMEM((L, _BPW), jnp.int32),
            pltpu.VMEM((_IDX_CH,), jnp.int32),
            pltpu.SemaphoreType.DMA,
            pltpu.SemaphoreType.DMA,
        ],
    )
    def prep_kernel(tT, xT, tR, idxP, buf, outv, slab, idxc, sem1, sem2):
        wid = lax.axis_index("s") * _NC + lax.axis_index("c")
        lane = lax.iota(jnp.int32, 16)
        d_idx = lane & 7
        roff = lane >> 3

        def chunk_body(k, carry):
            c = wid + k * _NW

            @pl.when(c < _NCHUNK)
            def _():
                r0 = c * _C
                cps = [pltpu.async_copy(
                    tT.at[pl.ds(d * V + r0, _C)], buf.at[d], sem1)
                    for d in range(D)]
                for cp in cps:
                    cp.wait()

                def g_body(g, carry2):
                    vals = plsc.load_gather(buf, [d_idx, roff + 2 * g])
                    outv[pl.ds(16 * g, 16)] = vals
                    return carry2

                lax.fori_loop(0, _C * D // 16, g_body, 0)
                pltpu.sync_copy(outv, tR.at[pl.ds(r0 * D, _C * D)])
            return carry

        lax.fori_loop(0, _CHUNK_ITERS, chunk_body, 0)

        b0 = wid * _BPW
        cps = [pltpu.async_copy(
            xT.at[pl.ds(l * B + b0, _BPW)], slab.at[l], sem2)
            for l in range(L)]
        for cp in cps:
            cp.wait()

        def blk_body(j, carry):
            def g_body(gg, carry2):
                g = j * (_IDX_CH // 16) + gg
                s_l = g >> 5
                ct = (g >> 3) & 3
                r = g & 7
                b_l = 8 * s_l + r
                vals = plsc.load_gather(slab, [16 * ct + lane, b_l + 0 * lane])
                idxc[pl.ds(16 * gg, 16)] = vals
                return carry2

            lax.fori_loop(0, _IDX_CH // 16, g_body, 0)
            pltpu.sync_copy(
                idxc, idxP.at[pl.ds(wid * _PER_W + j * _IDX_CH, _IDX_CH)])
            return carry

        lax.fori_loop(0, _PER_W // _IDX_CH, blk_body, 0)

    return prep_kernel(tableT_flat, xT_flat)


def _sc_gather(table, idx):
    mesh = plsc.VectorSubcoreMesh(core_axis_name="c", subcore_axis_name="s")

    @functools.partial(
        pl.kernel,
        mesh=mesh,
        compiler_params=pltpu.CompilerParams(use_tc_tiling_on_sc=False),
        out_type=jax.ShapeDtypeStruct((BL, D), jnp.float32),
        scratch_types=[
            pltpu.VMEM((_CH,), jnp.int32),
            pltpu.VMEM((_CH, D), jnp.float32),
            pltpu.SemaphoreType.DMA,
        ],
    )
    def gather_kernel(idx_hbm, table_hbm, out_hbm, idx_v, rows_v, sem):
        wid = lax.axis_index("s") * _NC + lax.axis_index("c")
        base = wid * _PER_W

        def body(i, carry):
            start = base + i * _CH
            pltpu.sync_copy(idx_hbm.at[pl.ds(start, _CH)], idx_v)
            pltpu.async_copy(table_hbm.at[idx_v], rows_v, sem).wait()
            pltpu.sync_copy(rows_v, out_hbm.at[pl.ds(start, _CH)])
            return carry

        lax.fori_loop(0, _NCH, body, 0)

    return gather_kernel(idx, table)


_BB = 512


def _mlp(x, emb4, E, W1, b1, W2, b2, W3, b3, W4, b4, W5, b5):
    def mlp_kernel(x_ref, emb_ref, E_ref, W1_ref, b1_ref, W2_ref, b2_ref,
                   W3_ref, b3_ref, W4_ref, b4_ref, W5_ref, b5_ref, out_ref):
        m = (x_ref[...] != 0).astype(jnp.float32)
        mask = jnp.dot(m, E_ref[...],
                       preferred_element_type=jnp.float32)
        e4 = emb_ref[...].reshape(_BB // 8, 4, 8, 128)
        h = None
        for ct in range(4):
            e = e4[:, ct, :, :].reshape(_BB, 128)
            e = e * mask[:, 128 * ct:128 * (ct + 1)]
            part = jnp.dot(e, W1_ref[pl.ds(128 * ct, 128), :],
                           preferred_element_type=jnp.float32)
            h = part if h is None else h + part
        h = jnp.maximum(h + b1_ref[...], 0.0)
        h = jnp.maximum(jnp.dot(h, W2_ref[...],
                                preferred_element_type=jnp.float32)
                        + b2_ref[...], 0.0)
        h = jnp.maximum(jnp.dot(h, W3_ref[...],
                                preferred_element_type=jnp.float32)
                        + b3_ref[...], 0.0)
        h = jnp.maximum(jnp.dot(h, W4_ref[...],
                                preferred_element_type=jnp.float32)
                        + b4_ref[...], 0.0)
        out_ref[...] = (jnp.dot(h, W5_ref[...],
                                preferred_element_type=jnp.float32)
                        + b5_ref[...])

    grid = (B // _BB,)
    full = lambda shape: pl.BlockSpec(shape, lambda i: tuple(0 for _ in shape))
    return pl.pallas_call(
        mlp_kernel,
        grid=grid,
        in_specs=[
            pl.BlockSpec((_BB, L), lambda i: (i, 0)),
            pl.BlockSpec((_BB * H // 128, 128), lambda i: (i, 0)),
            full((L, H)),
            full((512, 512)), full((1, 512)),
            full((512, 512)), full((1, 512)),
            full((512, 256)), full((1, 256)),
            full((256, 128)), full((1, 128)),
            full((128, 2)), full((1, 2)),
        ],
        out_specs=pl.BlockSpec((_BB, 2), lambda i: (i, 0)),
        out_shape=jax.ShapeDtypeStruct((B, 2), jnp.float32),
    )(x, emb4, E, W1, b1, W2, b2, W3, b3, W4, b4, W5, b5)


def kernel(x, table, W1, b1, W2, b2, W3, b3, W4, b4, W5, b5):
    tableR, idxP = _sc_prep(table.T.reshape(-1), x.T.reshape(-1))
    emb4 = _sc_gather(tableR.reshape(V, D), idxP).reshape(BL * D // 128, 128)
    E = jnp.repeat(jnp.eye(L, dtype=jnp.float32), D, axis=1)
    return _mlp(x, emb4, E,
                W1, b1.reshape(1, -1), W2, b2.reshape(1, -1),
                W3, b3.reshape(1, -1), W4, b4.reshape(1, -1),
                W5, b5.reshape(1, -1))


# speedup vs baseline: 1.5181x; 1.5181x over previous
import functools

import jax
import jax.numpy as jnp
from jax import lax
from jax.experimental import pallas as pl
from jax.experimental.pallas import tpu as pltpu
from jax.experimental.pallas import tpu_sc as plsc

B, L, V, D = 16384, 64, 1000000, 8
BL = B * L
H = L * D
NSLAB = B // 8


_NC, _NS = 2, 16
_NW = _NC * _NS
_PER_W = BL // _NW
_CH = 2048
_NCH = _PER_W // _CH

def _sc_gather(table, idx):
    mesh = plsc.VectorSubcoreMesh(core_axis_name="c", subcore_axis_name="s")

    @functools.partial(
        pl.kernel,
        mesh=mesh,
        compiler_params=pltpu.CompilerParams(use_tc_tiling_on_sc=False),
        out_type=jax.ShapeDtypeStruct((BL, D), jnp.float32),
        scratch_types=[
            pltpu.VMEM((_CH,), jnp.int32),
            pltpu.VMEM((_CH, D), jnp.float32),
            pltpu.SemaphoreType.DMA,
        ],
    )
    def gather_kernel(idx_hbm, table_hbm, out_hbm, idx_v, rows_v, sem):
        wid = lax.axis_index("s") * _NC + lax.axis_index("c")
        base = wid * _PER_W

        def body(i, carry):
            start = base + i * _CH
            pltpu.sync_copy(idx_hbm.at[pl.ds(start, _CH)], idx_v)
            pltpu.async_copy(table_hbm.at[idx_v], rows_v, sem).wait()
            pltpu.sync_copy(rows_v, out_hbm.at[pl.ds(start, _CH)])
            return carry

        lax.fori_loop(0, _NCH, body, 0)

    return gather_kernel(idx, table)


_BB = 512


def _mlp(x, emb4, E, W1, b1, W2, b2, W3, b3, W4, b4, W5, b5):
    bf16 = jnp.bfloat16

    def dotb(a, w_ref):
        return jnp.dot(a.astype(bf16), w_ref[...],
                       preferred_element_type=jnp.float32)

    def mlp_kernel(x_ref, emb_ref, E_ref, W1_ref, b1_ref, W2_ref, b2_ref,
                   W3_ref, b3_ref, W4_ref, b4_ref, W5_ref, b5_ref, out_ref):
        m = (x_ref[...] != 0).astype(bf16)
        mask = jnp.dot(m, E_ref[...],
                       preferred_element_type=jnp.float32)
        e4 = emb_ref[...].reshape(_BB // 8, 4, 8, 128)
        h = None
        for ct in range(4):
            e = e4[:, ct, :, :].reshape(_BB, 128)
            e = e * mask[:, 128 * ct:128 * (ct + 1)]
            part = jnp.dot(e.astype(bf16),
                           W1_ref[pl.ds(128 * ct, 128), :],
                           preferred_element_type=jnp.float32)
            h = part if h is None else h + part
        h = jnp.maximum(h + b1_ref[...], 0.0)
        h = jnp.maximum(dotb(h, W2_ref) + b2_ref[...], 0.0)
        h = jnp.maximum(dotb(h, W3_ref) + b3_ref[...], 0.0)
        h = jnp.maximum(dotb(h, W4_ref) + b4_ref[...], 0.0)
        out_ref[...] = dotb(h, W5_ref) + b5_ref[...]

    grid = (B // _BB,)
    full = lambda shape: pl.BlockSpec(shape, lambda i: tuple(0 for _ in shape))
    return pl.pallas_call(
        mlp_kernel,
        grid=grid,
        in_specs=[
            pl.BlockSpec((_BB, L), lambda i: (i, 0)),
            pl.BlockSpec((_BB * H // 128, 128), lambda i: (i, 0)),
            full((L, H)),
            full((512, 512)), full((1, 512)),
            full((512, 512)), full((1, 512)),
            full((512, 256)), full((1, 256)),
            full((256, 128)), full((1, 128)),
            full((128, 2)), full((1, 2)),
        ],
        out_specs=pl.BlockSpec((_BB, 2), lambda i: (i, 0)),
        out_shape=jax.ShapeDtypeStruct((B, 2), jnp.float32),
    )(x, emb4, E, W1, b1, W2, b2, W3, b3, W4, b4, W5, b5)


def kernel(x, table, W1, b1, W2, b2, W3, b3, W4, b4, W5, b5):
    idxP = x.T.reshape(4, 16, NSLAB, 8).transpose(2, 0, 3, 1).reshape(-1)
    emb4 = _sc_gather(table, idxP).reshape(BL * D // 128, 128)
    E = jnp.repeat(jnp.eye(L, dtype=jnp.bfloat16), D, axis=1)
    bf16 = jnp.bfloat16
    return _mlp(x, emb4, E,
                W1.astype(bf16), b1.reshape(1, -1),
                W2.astype(bf16), b2.reshape(1, -1),
                W3.astype(bf16), b3.reshape(1, -1),
                W4.astype(bf16), b4.reshape(1, -1),
                W5.astype(bf16), b5.reshape(1, -1))
